# baseline (speedup 1.0000x reference)
import functools

import jax
import jax.numpy as jnp
from jax import lax
from jax.experimental import pallas as pl
from jax.experimental.pallas import tpu as pltpu

BATCH = 16384
LABEL_DIM = 32
TEXT_DIM = 768

BM = 2048
NBLK = BATCH // BM
NBUF = 2


def _issue_rows(labels_smem, table_hbm, emb_v, sem, blk):
    base = blk * BM

    def issue(j, _):
        row = labels_smem[base + j]
        pltpu.make_async_copy(
            table_hbm.at[pl.ds(row, 1)], emb_v.at[pl.ds(j, 1)], sem
        ).start()
        return 0

    lax.fori_loop(0, BM, issue, 0, unroll=8)


def _body(labels_smem, table_hbm, w_ref, b_ref, out_ref, emb_v, sem):
    i = pl.program_id(0)

    @pl.when(i == 0)
    def _prologue():
        _issue_rows(labels_smem, table_hbm, emb_v.at[0], sem.at[0], 0)

    @pl.when(i + 1 < NBLK)
    def _next():
        _issue_rows(labels_smem, table_hbm, emb_v.at[(i + 1) % NBUF],
                    sem.at[(i + 1) % NBUF], i + 1)

    pltpu.make_async_copy(
        table_hbm.at[pl.ds(0, BM)], emb_v.at[i % NBUF], sem.at[i % NBUF]
    ).wait()
    w32 = emb_v[i % NBUF]
    lo = lax.bitcast_convert_type(w32 << 16, jnp.float32)
    hi = lax.bitcast_convert_type(w32 & jnp.int32(-65536), jnp.float32)
    emb = jnp.concatenate([lo, hi], axis=1)
    out_ref[...] = lax.dot_general(
        emb, w_ref[...],
        (((1,), (1,)), ((), ())),
        preferred_element_type=jnp.float32,
    ) + b_ref[...]


def kernel(labels, label_emb, W, b):
    table_bf = label_emb.astype(jnp.bfloat16).reshape(1000000, 16, 2)
    table_i32 = lax.bitcast_convert_type(table_bf, jnp.int32)
    w_perm = jnp.concatenate([W[:, 0::2], W[:, 1::2]], axis=1)
    b2d = b.reshape(1, TEXT_DIM)
    grid_spec = pltpu.PrefetchScalarGridSpec(
        num_scalar_prefetch=1,
        grid=(NBLK,),
        in_specs=[
            pl.BlockSpec(memory_space=pl.ANY),
            pl.BlockSpec((TEXT_DIM, LABEL_DIM), lambda i, *_: (0, 0)),
            pl.BlockSpec((1, TEXT_DIM), lambda i, *_: (0, 0)),
        ],
        out_specs=pl.BlockSpec((BM, TEXT_DIM), lambda i, *_: (i, 0)),
        scratch_shapes=[
            pltpu.VMEM((NBUF, BM, LABEL_DIM // 2), jnp.int32),
            pltpu.SemaphoreType.DMA((NBUF,)),
        ],
    )
    out = pl.pallas_call(
        _body,
        grid_spec=grid_spec,
        out_shape=jax.ShapeDtypeStruct((BATCH, TEXT_DIM), jnp.float32),
    )(labels, table_i32, w_perm, b2d)
    return out


# speedup vs baseline: 2.6490x; 2.6490x over previous
import functools

import jax
import jax.numpy as jnp
from jax import lax
from jax.experimental import pallas as pl
from jax.experimental.pallas import tpu as pltpu

BATCH = 16384
LABEL_DIM = 32
TEXT_DIM = 768

BM = 2048
NBLK = BATCH // BM
NBUF = 2


def _issue_rows(labels_smem, table_hbm, emb_v, sem, blk):
    base = blk * BM

    def issue(j, _):
        row = labels_smem[base + j]
        pltpu.make_async_copy(
            table_hbm.at[pl.ds(row, 1)], emb_v.at[pl.ds(j, 1)], sem
        ).start()
        return 0

    lax.fori_loop(0, BM, issue, 0, unroll=8)


def _body(labels_smem, table_hbm, w_ref, b_ref, out_ref, emb_v, sem):
    i = pl.program_id(0)

    @pl.when(i == 0)
    def _prologue():
        _issue_rows(labels_smem, table_hbm, emb_v.at[0], sem.at[0], 0)

    @pl.when(i + 1 < NBLK)
    def _next():
        _issue_rows(labels_smem, table_hbm, emb_v.at[(i + 1) % NBUF],
                    sem.at[(i + 1) % NBUF], i + 1)

    pltpu.make_async_copy(
        table_hbm.at[pl.ds(0, BM)], emb_v.at[i % NBUF], sem.at[i % NBUF]
    ).wait()
    out_ref[...] = lax.dot_general(
        emb_v[i % NBUF], w_ref[...],
        (((1,), (1,)), ((), ())),
        preferred_element_type=jnp.float32,
    ) + b_ref[...]


def kernel(labels, label_emb, W, b):
    b2d = b.reshape(1, TEXT_DIM)
    grid_spec = pltpu.PrefetchScalarGridSpec(
        num_scalar_prefetch=1,
        grid=(NBLK,),
        in_specs=[
            pl.BlockSpec(memory_space=pl.ANY),
            pl.BlockSpec((TEXT_DIM, LABEL_DIM), lambda i, *_: (0, 0)),
            pl.BlockSpec((1, TEXT_DIM), lambda i, *_: (0, 0)),
        ],
        out_specs=pl.BlockSpec((BM, TEXT_DIM), lambda i, *_: (i, 0)),
        scratch_shapes=[
            pltpu.VMEM((NBUF, BM, LABEL_DIM), jnp.float32),
            pltpu.SemaphoreType.DMA((NBUF,)),
        ],
    )
    out = pl.pallas_call(
        _body,
        grid_spec=grid_spec,
        out_shape=jax.ShapeDtypeStruct((BATCH, TEXT_DIM), jnp.float32),
    )(labels, label_emb, W, b2d)
    return out
